# baseline (speedup 1.0000x reference)
import jax
import jax.numpy as jnp
from jax.experimental import pallas as pl
from jax.experimental.pallas import tpu as pltpu

N = 320000
R_IN = 128
R_OUT = 256
C_OUT = 256
NS = 10000

B = 2560
NB = N // B
C = 512
NC = B // C
K = 32
KP = K + 8


def _seg_kernel(s0_ref, smax_ref, ids_ref, x_ref, w1t_ref, b1_ref,
                sums_ref, counts_ref):
    i = pl.program_id(0)

    @pl.when(i == 0)
    def _():
        sums_ref[...] = jnp.zeros_like(sums_ref)
        counts_ref[...] = jnp.zeros_like(counts_ref)

    xb = x_ref[...].astype(jnp.bfloat16)
    h = jnp.dot(xb, w1t_ref[...], preferred_element_type=jnp.float32)
    h = jnp.maximum(h + b1_ref[...], 0.0)
    hb = h.astype(jnp.bfloat16)

    ids = ids_ref[0]

    for c in range(NC):
        ids_c = ids[:, c * C:(c + 1) * C]
        hb_c = hb[c * C:(c + 1) * C, :]
        s0 = s0_ref[i * NC + c]
        smax = smax_ref[i * NC + c]
        nwin = (smax - s0) // K + 1

        def win(j, carry, s0=s0, ids_c=ids_c, hb_c=hb_c):
            base = s0 + j * K
            wb = jnp.minimum((base // 8) * 8, NS - KP)
            pos = ids_c - wb
            rel = ids_c - base
            row = jax.lax.broadcasted_iota(jnp.int32, (KP, C), 0)
            oh = (row == pos) & (rel >= 0) & (rel < K)
            ohf = oh.astype(jnp.bfloat16)
            ls = jax.lax.dot_general(ohf, hb_c, (((1,), (0,)), ((), ())),
                                     preferred_element_type=jnp.float32)
            lc = jnp.sum(oh.astype(jnp.float32), axis=1, keepdims=True)
            sums_ref[pl.ds(wb, KP), :] += ls
            counts_ref[pl.ds(wb, KP), :] += lc
            return carry

        jax.lax.fori_loop(0, nwin, win, 0)


def _head_kernel(sums_ref, counts_ref, w2t_ref, b2_ref, out_ref):
    mean = sums_ref[...] / jnp.maximum(counts_ref[...], 1.0)
    out = jnp.dot(mean, w2t_ref[...], preferred_element_type=jnp.float32)
    out_ref[...] = jnp.maximum(out + b2_ref[...], 0.0)


def kernel(x, batch_index, W1, b1, W2, b2):
    bi = batch_index.astype(jnp.int32)
    s0 = bi[::C]
    smax = bi[C - 1::C]
    ids3 = bi.reshape(NB, 1, B)

    grid_spec = pltpu.PrefetchScalarGridSpec(
        num_scalar_prefetch=2,
        grid=(NB,),
        in_specs=[
            pl.BlockSpec((1, 1, B), lambda i, *_: (i, 0, 0)),
            pl.BlockSpec((B, R_IN), lambda i, *_: (i, 0)),
            pl.BlockSpec((R_IN, R_OUT), lambda i, *_: (0, 0)),
            pl.BlockSpec((1, R_OUT), lambda i, *_: (0, 0)),
        ],
        out_specs=[
            pl.BlockSpec((NS, R_OUT), lambda i, *_: (0, 0)),
            pl.BlockSpec((NS, 1), lambda i, *_: (0, 0)),
        ],
    )
    sums, counts = pl.pallas_call(
        _seg_kernel,
        grid_spec=grid_spec,
        out_shape=[
            jax.ShapeDtypeStruct((NS, R_OUT), jnp.float32),
            jax.ShapeDtypeStruct((NS, 1), jnp.float32),
        ],
    )(s0, smax, ids3, x, W1.T.astype(jnp.bfloat16), b1.reshape(1, R_OUT))

    R = 2000
    out = pl.pallas_call(
        _head_kernel,
        grid=(NS // R,),
        in_specs=[
            pl.BlockSpec((R, R_OUT), lambda i: (i, 0)),
            pl.BlockSpec((R, 1), lambda i: (i, 0)),
            pl.BlockSpec((R_OUT, C_OUT), lambda i: (0, 0)),
            pl.BlockSpec((1, C_OUT), lambda i: (0, 0)),
        ],
        out_specs=pl.BlockSpec((R, C_OUT), lambda i: (i, 0)),
        out_shape=jax.ShapeDtypeStruct((NS, C_OUT), jnp.float32),
    )(sums, counts, W2.T, b2.reshape(1, C_OUT))
    return out


# speedup vs baseline: 1.2001x; 1.2001x over previous
import jax
import jax.numpy as jnp
from jax.experimental import pallas as pl
from jax.experimental.pallas import tpu as pltpu

N = 320000
R_IN = 128
R_OUT = 256
C_OUT = 256
NS = 10000

B = 2560
NB = N // B
KP = 104
NSP = 10112


def _seg_kernel(s0_ref, smax_ref, ids_ref, x_ref, w1t_ref, b1_ref,
                sums_ref, counts_ref):
    i = pl.program_id(0)

    @pl.when(i == 0)
    def _():
        sums_ref[...] = jnp.zeros_like(sums_ref)
        counts_ref[...] = jnp.zeros_like(counts_ref)

    xb = x_ref[...].astype(jnp.bfloat16)
    h = jnp.dot(xb, w1t_ref[...], preferred_element_type=jnp.float32)
    hb = jnp.maximum(h + b1_ref[...], 0).astype(jnp.bfloat16)

    ids = ids_ref[0]
    a0 = (s0_ref[i] // 8) * 8
    smax = smax_ref[i]
    nwin = (smax - a0) // KP + 1

    def win(j, carry):
        base = a0 + j * KP
        pos = ids - base
        row = jax.lax.broadcasted_iota(jnp.int32, (KP, B), 0)
        oh = row == pos
        ohf = oh.astype(jnp.bfloat16)
        ls = jax.lax.dot_general(ohf, hb, (((1,), (0,)), ((), ())),
                                 preferred_element_type=jnp.float32)
        lc = jnp.sum(oh.astype(jnp.float32), axis=1, keepdims=True)
        sums_ref[pl.ds(base, KP), :] += ls
        counts_ref[pl.ds(base, KP), :] += lc
        return carry

    jax.lax.fori_loop(0, nwin, win, 0)


def _head_kernel(sums_ref, counts_ref, w2t_ref, b2_ref, out_ref):
    mean = sums_ref[...] / jnp.maximum(counts_ref[...], 1.0)
    out = jnp.dot(mean, w2t_ref[...], preferred_element_type=jnp.float32)
    out_ref[...] = jnp.maximum(out + b2_ref[...], 0.0)


def kernel(x, batch_index, W1, b1, W2, b2):
    bi = batch_index.astype(jnp.int32)
    s0 = bi[::B]
    smax = bi[B - 1::B]
    ids3 = bi.reshape(NB, 1, B)

    grid_spec = pltpu.PrefetchScalarGridSpec(
        num_scalar_prefetch=2,
        grid=(NB,),
        in_specs=[
            pl.BlockSpec((1, 1, B), lambda i, *_: (i, 0, 0)),
            pl.BlockSpec((B, R_IN), lambda i, *_: (i, 0)),
            pl.BlockSpec((R_IN, R_OUT), lambda i, *_: (0, 0)),
            pl.BlockSpec((1, R_OUT), lambda i, *_: (0, 0)),
        ],
        out_specs=[
            pl.BlockSpec((NSP, R_OUT), lambda i, *_: (0, 0)),
            pl.BlockSpec((NSP, 1), lambda i, *_: (0, 0)),
        ],
    )
    sums, counts = pl.pallas_call(
        _seg_kernel,
        grid_spec=grid_spec,
        out_shape=[
            jax.ShapeDtypeStruct((NSP, R_OUT), jnp.float32),
            jax.ShapeDtypeStruct((NSP, 1), jnp.float32),
        ],
    )(s0, smax, ids3, x, W1.T.astype(jnp.bfloat16), b1.reshape(1, R_OUT))

    R = 2000
    out = pl.pallas_call(
        _head_kernel,
        grid=(NS // R,),
        in_specs=[
            pl.BlockSpec((R, R_OUT), lambda i: (i, 0)),
            pl.BlockSpec((R, 1), lambda i: (i, 0)),
            pl.BlockSpec((R_OUT, C_OUT), lambda i: (0, 0)),
            pl.BlockSpec((1, C_OUT), lambda i: (0, 0)),
        ],
        out_specs=pl.BlockSpec((R, C_OUT), lambda i: (i, 0)),
        out_shape=jax.ShapeDtypeStruct((NS, C_OUT), jnp.float32),
    )(sums, counts, W2.T, b2.reshape(1, C_OUT))
    return out


# speedup vs baseline: 1.4199x; 1.1832x over previous
import jax
import jax.numpy as jnp
from jax.experimental import pallas as pl
from jax.experimental.pallas import tpu as pltpu

N = 320000
R_IN = 128
R_OUT = 256
C_OUT = 256
NS = 10000

B = 4000
NB = N // B
KP = 136
NSP = 10128


def _seg_kernel(s0_ref, smax_ref, ids_ref, x_ref, w1t_ref, b1_ref,
                sums_ref, counts_ref):
    i = pl.program_id(0)

    @pl.when(i == 0)
    def _():
        sums_ref[...] = jnp.zeros_like(sums_ref)
        counts_ref[...] = jnp.zeros_like(counts_ref)

    xb = x_ref[...].astype(jnp.bfloat16)
    h = jnp.dot(xb, w1t_ref[...], preferred_element_type=jnp.float32)
    hb = jnp.maximum(h + b1_ref[...], 0).astype(jnp.bfloat16)

    ids = ids_ref[0]
    a0 = (s0_ref[i] // 8) * 8
    smax = smax_ref[i]
    nwin = (smax - a0) // KP + 1

    def win(j, carry):
        base = a0 + j * KP
        pos = ids - base
        row = jax.lax.broadcasted_iota(jnp.int32, (KP, B), 0)
        oh = row == pos
        ohf = oh.astype(jnp.bfloat16)
        ls = jax.lax.dot_general(ohf, hb, (((1,), (0,)), ((), ())),
                                 preferred_element_type=jnp.float32)
        lc = jnp.sum(oh.astype(jnp.float32), axis=1, keepdims=True)
        sums_ref[pl.ds(base, KP), :] += ls
        counts_ref[pl.ds(base, KP), :] += lc
        return carry

    jax.lax.fori_loop(0, nwin, win, 0)


def _head_kernel(sums_ref, counts_ref, w2t_ref, b2_ref, out_ref):
    mean = sums_ref[...] / jnp.maximum(counts_ref[...], 1.0)
    out = jnp.dot(mean, w2t_ref[...], preferred_element_type=jnp.float32)
    out_ref[...] = jnp.maximum(out + b2_ref[...], 0.0)


def kernel(x, batch_index, W1, b1, W2, b2):
    bi = batch_index.astype(jnp.int32)
    s0 = bi[::B]
    smax = bi[B - 1::B]
    ids3 = bi.reshape(NB, 1, B)

    grid_spec = pltpu.PrefetchScalarGridSpec(
        num_scalar_prefetch=2,
        grid=(NB,),
        in_specs=[
            pl.BlockSpec((1, 1, B), lambda i, *_: (i, 0, 0)),
            pl.BlockSpec((B, R_IN), lambda i, *_: (i, 0)),
            pl.BlockSpec((R_IN, R_OUT), lambda i, *_: (0, 0)),
            pl.BlockSpec((1, R_OUT), lambda i, *_: (0, 0)),
        ],
        out_specs=[
            pl.BlockSpec((NSP, R_OUT), lambda i, *_: (0, 0)),
            pl.BlockSpec((NSP, 1), lambda i, *_: (0, 0)),
        ],
    )
    sums, counts = pl.pallas_call(
        _seg_kernel,
        grid_spec=grid_spec,
        out_shape=[
            jax.ShapeDtypeStruct((NSP, R_OUT), jnp.float32),
            jax.ShapeDtypeStruct((NSP, 1), jnp.float32),
        ],
    )(s0, smax, ids3, x, W1.T.astype(jnp.bfloat16), b1.reshape(1, R_OUT))

    R = 2000
    out = pl.pallas_call(
        _head_kernel,
        grid=(NS // R,),
        in_specs=[
            pl.BlockSpec((R, R_OUT), lambda i: (i, 0)),
            pl.BlockSpec((R, 1), lambda i: (i, 0)),
            pl.BlockSpec((R_OUT, C_OUT), lambda i: (0, 0)),
            pl.BlockSpec((1, C_OUT), lambda i: (0, 0)),
        ],
        out_specs=pl.BlockSpec((R, C_OUT), lambda i: (i, 0)),
        out_shape=jax.ShapeDtypeStruct((NS, C_OUT), jnp.float32),
    )(sums, counts, W2.T, b2.reshape(1, C_OUT))
    return out
